# Initial kernel scaffold; baseline (speedup 1.0000x reference)
import jax
import jax.numpy as jnp
from jax.experimental import pallas as pl


def kernel(x, edge_src, edge_dst, edge_val, embed, W1, B1, W2, B2, W3, B3, g1, be1, g2, be2, g3, be3):
    raise NotImplementedError("write your pallas kernel here")


# speedup vs baseline: 3.2924x; 3.2924x over previous
import functools

import jax
import jax.numpy as jnp
from jax import lax
from jax.experimental import pallas as pl
from jax.experimental.pallas import tpu as pltpu
from jax.experimental.pallas import tpu_sc as plsc

_N = 10000
_E = 320000
_D = 128
_B = 16384

_NC = 2
_NS = 16
_NW = _NC * _NS
_L = 16

_CH = 128
_NCH = -(-(_E // _NW) // _CH)
_EW = _NCH * _CH
_EPAD = _EW * _NW
_NP = 10240
_RPT = _NP // _NS


def _bcast_lane(v16, k):
  idx = jnp.full((_L,), k, dtype=jnp.int32)
  return lax.gather(
      v16,
      idx[:, None],
      lax.GatherDimensionNumbers(
          offset_dims=(), collapsed_slice_dims=(0,), start_index_map=(0,)),
      (1,),
      mode=lax.GatherScatterMode.PROMISE_IN_BOUNDS)


def _propagate(h, src, dst, val):
  mesh = plsc.VectorSubcoreMesh(core_axis_name="c", subcore_axis_name="s")

  @functools.partial(
      pl.kernel,
      out_type=jax.ShapeDtypeStruct((_NC * _NP, _D), jnp.float32),
      mesh=mesh,
      scratch_types=[
          pltpu.VMEM((_CH,), jnp.int32),
          pltpu.VMEM((_CH,), jnp.int32),
          pltpu.VMEM((_CH,), jnp.float32),
          pltpu.VMEM((_CH, _D), jnp.float32),
          pltpu.VMEM_SHARED((_NP, _D), jnp.float32),
          pltpu.SemaphoreType.DMA,
      ],
  )
  def prop(h_hbm, src_hbm, dst_hbm, val_hbm, out_hbm,
           src_v, dst_v, val_v, rows_v, acc, sem):
    cid = lax.axis_index("c")
    sid = lax.axis_index("s")
    wid = sid * _NC + cid

    zero16 = jnp.zeros((_L,), jnp.float32)

    @pl.loop(0, _CH)
    def _zero(r):
      for j in range(_D // _L):
        rows_v[r, pl.ds(j * _L, _L)] = zero16

    row0 = sid * _RPT
    for t in range(_RPT // _CH):
      pltpu.sync_copy(rows_v, acc.at[pl.ds(row0 + t * _CH, _CH)])
    plsc.subcore_barrier()

    @pl.loop(0, _NCH)
    def _chunk(i):
      base = (wid * _NCH + i) * _CH
      pltpu.sync_copy(src_hbm.at[pl.ds(base, _CH)], src_v)
      pltpu.sync_copy(val_hbm.at[pl.ds(base, _CH)], val_v)
      pltpu.sync_copy(dst_hbm.at[pl.ds(base, _CH)], dst_v)
      pltpu.async_copy(h_hbm.at[src_v], rows_v, sem).wait()

      @pl.loop(0, _CH // _L)
      def _grp(g):
        val16 = val_v[pl.ds(g * _L, _L)]
        for k in range(_L):
          vk = _bcast_lane(val16, k)
          r = g * _L + k
          for j in range(_D // _L):
            sl = pl.ds(j * _L, _L)
            rows_v[r, sl] = rows_v[r, sl] * vk

      pltpu.sync_copy(rows_v, acc.at[dst_v], add=True)

    plsc.subcore_barrier()
    pltpu.sync_copy(acc.at[pl.ds(row0, _RPT)],
                    out_hbm.at[pl.ds(cid * _NP + row0, _RPT)])

  out = prop(h, src, dst, val)
  return out


def _gather_rows(full, idx, width):
  mesh = plsc.VectorSubcoreMesh(core_axis_name="c", subcore_axis_name="s")
  per_w = _B // _NW
  gch = 128
  nit = per_w // gch

  @functools.partial(
      pl.kernel,
      out_type=jax.ShapeDtypeStruct((_B, width), jnp.float32),
      mesh=mesh,
      scratch_types=[
          pltpu.VMEM((gch,), jnp.int32),
          pltpu.VMEM((gch, width), jnp.float32),
          pltpu.SemaphoreType.DMA,
      ],
  )
  def gat(full_hbm, idx_hbm, out_hbm, idx_v, rows_v, sem):
    cid = lax.axis_index("c")
    sid = lax.axis_index("s")
    wid = sid * _NC + cid
    for i in range(nit):
      base = wid * per_w + i * gch
      pltpu.sync_copy(idx_hbm.at[pl.ds(base, gch)], idx_v)
      one16 = jnp.full((_L,), 1, dtype=jnp.int32)
      for j in range(gch // _L):
        sl = pl.ds(j * _L, _L)
        idx_v[sl] = idx_v[sl] - one16
      pltpu.async_copy(full_hbm.at[idx_v], rows_v, sem).wait()
      pltpu.sync_copy(rows_v, out_hbm.at[pl.ds(base, gch)])

  return gat(full, idx)


def _mm_bias(x, W, b):
  m, k = x.shape
  n = W.shape[1]
  bm = 1000

  def body(x_ref, w_ref, b_ref, o_ref):
    o_ref[...] = jnp.dot(x_ref[...], w_ref[...],
                         preferred_element_type=jnp.float32) + b_ref[...]

  return pl.pallas_call(
      body,
      grid=(m // bm,),
      in_specs=[
          pl.BlockSpec((bm, k), lambda i: (i, 0)),
          pl.BlockSpec((k, n), lambda i: (0, 0)),
          pl.BlockSpec((1, n), lambda i: (0, 0)),
      ],
      out_specs=pl.BlockSpec((bm, n), lambda i: (i, 0)),
      out_shape=jax.ShapeDtypeStruct((m, n), jnp.float32),
  )(x, W, b.reshape(1, n))


def _combine_ln(parts, g, be):
  bm = 1000

  def body(p0_ref, p1_ref, g_ref, b_ref, o_ref):
    h = jax.nn.relu(p0_ref[...] + p1_ref[...])
    mu = jnp.mean(h, axis=-1, keepdims=True)
    var = jnp.mean((h - mu) * (h - mu), axis=-1, keepdims=True)
    o_ref[...] = (h - mu) * lax.rsqrt(var + 1e-5) * g_ref[...] + b_ref[...]

  return pl.pallas_call(
      body,
      grid=(_N // bm,),
      in_specs=[
          pl.BlockSpec((bm, _D), lambda i: (i, 0)),
          pl.BlockSpec((bm, _D), lambda i: (i, 0)),
          pl.BlockSpec((1, _D), lambda i: (0, 0)),
          pl.BlockSpec((1, _D), lambda i: (0, 0)),
      ],
      out_specs=pl.BlockSpec((bm, _D), lambda i: (i, 0)),
      out_shape=jax.ShapeDtypeStruct((_N, _D), jnp.float32),
  )(parts[:_N], parts[_NP:_NP + _N], g.reshape(1, _D), be.reshape(1, _D))


def kernel(x, edge_src, edge_dst, edge_val, embed, W1, B1, W2, B2, W3, B3,
           g1, be1, g2, be2, g3, be3):
  pad = _EPAD - _E
  src = jnp.pad(edge_src, (0, pad))
  dst = jnp.pad(edge_dst, (0, pad))
  val = jnp.pad(edge_val, (0, pad))

  h1 = _mm_bias(embed, W1, B1)
  e1 = _combine_ln(_propagate(h1, src, dst, val), g1, be1)

  h2 = _mm_bias(e1, W2, B2)
  e2 = _combine_ln(_propagate(h2, src, dst, val), g2, be2)

  e2c = jnp.concatenate((e1, e2), axis=1)
  h3 = _mm_bias(e2c, W3, B3)
  e3 = _combine_ln(_propagate(h3, src, dst, val), g3, be3)

  full = jnp.concatenate((e2c, e3), axis=1)
  out = _gather_rows(full, x, 3 * _D)
  recon_loss = jnp.zeros((1,), jnp.float32)
  return (out, recon_loss)
